# Initial kernel scaffold; baseline (speedup 1.0000x reference)
import functools

import jax
import jax.numpy as jnp
from jax import lax
from jax.experimental import pallas as pl
from jax.experimental.pallas import tpu as pltpu


def _router_body(x_ref, w1_ref, b1_ref, w2_ref, b2_ref, out_ref, idx_ref,
                 acc_ref, *, k_top, n_e, bn, prec1, prec2):
    j = pl.program_id(1)
    nj = pl.num_programs(1)

    h = lax.dot_general(x_ref[...], w1_ref[...], (((1,), (1,)), ((), ())),
                        preferred_element_type=jnp.float32, precision=prec1)
    h = jnp.maximum(h + b1_ref[...], 0.0)
    part = lax.dot_general(h, w2_ref[...], (((1,), (1,)), ((), ())),
                           preferred_element_type=jnp.float32, precision=prec2)

    @pl.when(j == 0)
    def _init():
        acc_ref[...] = part

    @pl.when(j > 0)
    def _accum():
        acc_ref[...] += part

    @pl.when(j == nj - 1)
    def _epilogue():
        logits = acc_ref[...] + b2_ref[...]
        e_iota = lax.broadcasted_iota(jnp.int32, (bn, n_e), 1)
        r_iota = lax.broadcasted_iota(jnp.int32, (bn, k_top), 1)
        work = logits
        sel = jnp.zeros((bn, n_e), jnp.bool_)
        idx_out = jnp.zeros((bn, k_top), jnp.int32)
        top0 = None
        for k in range(k_top):
            m = jnp.max(work, axis=1, keepdims=True)
            hit = work == m
            idxk = jnp.min(jnp.where(hit, e_iota, n_e), axis=1, keepdims=True)
            picked = e_iota == idxk
            work = jnp.where(picked, -jnp.inf, work)
            sel = jnp.logical_or(sel, picked)
            idx_out = jnp.where(r_iota == k, idxk, idx_out)
            if k == 0:
                top0 = m
        ex = jnp.where(sel, jnp.exp(logits - top0), 0.0)
        out_ref[...] = ex / jnp.sum(ex, axis=1, keepdims=True)
        idx_ref[...] = idx_out


@jax.jit
def kernel(x, W1, b1, W2, b2):
    n, d = x.shape
    h_dim = W1.shape[0]
    n_e = W2.shape[0]
    k_top = 8
    bn = min(512, n)
    bh = min(512, h_dim)
    assert n % bn == 0 and h_dim % bh == 0

    b1r = b1.reshape(1, h_dim)
    b2r = b2.reshape(1, n_e)

    body = functools.partial(
        _router_body, k_top=k_top, n_e=n_e, bn=bn,
        prec1=lax.Precision.HIGHEST, prec2=lax.Precision.HIGHEST)

    out, idx = pl.pallas_call(
        body,
        grid=(n // bn, h_dim // bh),
        in_specs=[
            pl.BlockSpec((bn, d), lambda i, j: (i, 0)),
            pl.BlockSpec((bh, d), lambda i, j: (j, 0)),
            pl.BlockSpec((1, bh), lambda i, j: (0, j)),
            pl.BlockSpec((n_e, bh), lambda i, j: (0, j)),
            pl.BlockSpec((1, n_e), lambda i, j: (0, 0)),
        ],
        out_specs=[
            pl.BlockSpec((bn, n_e), lambda i, j: (i, 0)),
            pl.BlockSpec((bn, k_top), lambda i, j: (i, 0)),
        ],
        out_shape=[
            jax.ShapeDtypeStruct((n, n_e), jnp.float32),
            jax.ShapeDtypeStruct((n, k_top), jnp.int32),
        ],
        scratch_shapes=[pltpu.VMEM((bn, n_e), jnp.float32)],
        compiler_params=pltpu.CompilerParams(
            dimension_semantics=("parallel", "arbitrary")),
    )(x, W1, b1r, W2, b2r)
    return (out, idx)


# speedup vs baseline: 1.4665x; 1.4665x over previous
import functools

import jax
import jax.numpy as jnp
from jax import lax
from jax.experimental import pallas as pl
from jax.experimental.pallas import tpu as pltpu


def _router_body(x_ref, w1_ref, b1_ref, w2_ref, b2_ref, out_ref, idx_ref,
                 acc_ref, *, k_top, n_e, bn, prec1, prec2):
    j = pl.program_id(1)
    nj = pl.num_programs(1)

    h = lax.dot_general(x_ref[...], w1_ref[...], (((1,), (1,)), ((), ())),
                        preferred_element_type=jnp.float32, precision=prec1)
    h = jnp.maximum(h + b1_ref[...], 0.0)
    part = lax.dot_general(h, w2_ref[...], (((1,), (1,)), ((), ())),
                           preferred_element_type=jnp.float32, precision=prec2)

    @pl.when(j == 0)
    def _init():
        acc_ref[...] = part

    @pl.when(j > 0)
    def _accum():
        acc_ref[...] += part

    @pl.when(j == nj - 1)
    def _epilogue():
        logits = acc_ref[...] + b2_ref[...]
        e_iota = lax.broadcasted_iota(jnp.int32, (bn, n_e), 1)
        r_iota = lax.broadcasted_iota(jnp.int32, (bn, k_top), 1)
        work = logits
        sel = jnp.zeros((bn, n_e), jnp.bool_)
        idx_out = jnp.zeros((bn, k_top), jnp.int32)
        top0 = None
        for k in range(k_top):
            m = jnp.max(work, axis=1, keepdims=True)
            hit = work == m
            idxk = jnp.min(jnp.where(hit, e_iota, n_e), axis=1, keepdims=True)
            picked = e_iota == idxk
            work = jnp.where(picked, -jnp.inf, work)
            sel = jnp.logical_or(sel, picked)
            idx_out = jnp.where(r_iota == k, idxk, idx_out)
            if k == 0:
                top0 = m
        ex = jnp.where(sel, jnp.exp(logits - top0), 0.0)
        out_ref[...] = ex / jnp.sum(ex, axis=1, keepdims=True)
        idx_ref[...] = idx_out


@jax.jit
def kernel(x, W1, b1, W2, b2):
    n, d = x.shape
    h_dim = W1.shape[0]
    n_e = W2.shape[0]
    k_top = 8
    bn = min(512, n)
    bh = min(512, h_dim)
    assert n % bn == 0 and h_dim % bh == 0

    b1r = b1.reshape(1, h_dim)
    b2r = b2.reshape(1, n_e)

    body = functools.partial(
        _router_body, k_top=k_top, n_e=n_e, bn=bn,
        prec1=lax.Precision.DEFAULT, prec2=lax.Precision.DEFAULT)

    out, idx = pl.pallas_call(
        body,
        grid=(n // bn, h_dim // bh),
        in_specs=[
            pl.BlockSpec((bn, d), lambda i, j: (i, 0)),
            pl.BlockSpec((bh, d), lambda i, j: (j, 0)),
            pl.BlockSpec((1, bh), lambda i, j: (0, j)),
            pl.BlockSpec((n_e, bh), lambda i, j: (0, j)),
            pl.BlockSpec((1, n_e), lambda i, j: (0, 0)),
        ],
        out_specs=[
            pl.BlockSpec((bn, n_e), lambda i, j: (i, 0)),
            pl.BlockSpec((bn, k_top), lambda i, j: (i, 0)),
        ],
        out_shape=[
            jax.ShapeDtypeStruct((n, n_e), jnp.float32),
            jax.ShapeDtypeStruct((n, k_top), jnp.int32),
        ],
        scratch_shapes=[pltpu.VMEM((bn, n_e), jnp.float32)],
        compiler_params=pltpu.CompilerParams(
            dimension_semantics=("parallel", "arbitrary")),
    )(x, W1, b1r, W2, b2r)
    return (out, idx)


# speedup vs baseline: 1.7870x; 1.2186x over previous
import functools

import jax
import jax.numpy as jnp
from jax import lax
from jax.experimental import pallas as pl
from jax.experimental.pallas import tpu as pltpu


def _router_body(x_ref, w1_ref, b1_ref, w2_ref, b2_ref, out_ref, idx_ref,
                 acc_ref, *, k_top, n_e, bn, prec1, prec2):
    j = pl.program_id(1)
    nj = pl.num_programs(1)

    h = lax.dot_general(x_ref[...], w1_ref[...], (((1,), (1,)), ((), ())),
                        preferred_element_type=jnp.float32, precision=prec1)
    h = jnp.maximum(h + b1_ref[...], 0.0)
    part = lax.dot_general(h, w2_ref[...], (((1,), (1,)), ((), ())),
                           preferred_element_type=jnp.float32, precision=prec2)

    @pl.when(j == 0)
    def _init():
        acc_ref[...] = part

    @pl.when(j > 0)
    def _accum():
        acc_ref[...] += part

    @pl.when(j == nj - 1)
    def _epilogue():
        logits = acc_ref[...] + b2_ref[...]
        e_iota = lax.broadcasted_iota(jnp.int32, (bn, n_e), 1)
        r_iota = lax.broadcasted_iota(jnp.int32, (bn, k_top), 1)
        work = logits
        sel = jnp.zeros((bn, n_e), jnp.bool_)
        idx_out = jnp.zeros((bn, k_top), jnp.int32)
        top0 = None
        for k in range(k_top):
            m = jnp.max(work, axis=1, keepdims=True)
            hit = work == m
            idxk = jnp.min(jnp.where(hit, e_iota, n_e), axis=1, keepdims=True)
            picked = e_iota == idxk
            work = jnp.where(picked, -jnp.inf, work)
            sel = jnp.logical_or(sel, picked)
            idx_out = jnp.where(r_iota == k, idxk, idx_out)
            if k == 0:
                top0 = m
        ex = jnp.where(sel, jnp.exp(logits - top0), 0.0)
        out_ref[...] = ex / jnp.sum(ex, axis=1, keepdims=True)
        idx_ref[...] = idx_out


@jax.jit
def kernel(x, W1, b1, W2, b2):
    n, d = x.shape
    h_dim = W1.shape[0]
    n_e = W2.shape[0]
    k_top = 8
    bn = min(1024, n)
    bh = min(512, h_dim)
    assert n % bn == 0 and h_dim % bh == 0

    b1r = b1.reshape(1, h_dim)
    b2r = b2.reshape(1, n_e)

    body = functools.partial(
        _router_body, k_top=k_top, n_e=n_e, bn=bn,
        prec1=lax.Precision.DEFAULT, prec2=lax.Precision.DEFAULT)

    out, idx = pl.pallas_call(
        body,
        grid=(n // bn, h_dim // bh),
        in_specs=[
            pl.BlockSpec((bn, d), lambda i, j: (i, 0)),
            pl.BlockSpec((bh, d), lambda i, j: (j, 0)),
            pl.BlockSpec((1, bh), lambda i, j: (0, j)),
            pl.BlockSpec((n_e, bh), lambda i, j: (0, j)),
            pl.BlockSpec((1, n_e), lambda i, j: (0, 0)),
        ],
        out_specs=[
            pl.BlockSpec((bn, n_e), lambda i, j: (i, 0)),
            pl.BlockSpec((bn, k_top), lambda i, j: (i, 0)),
        ],
        out_shape=[
            jax.ShapeDtypeStruct((n, n_e), jnp.float32),
            jax.ShapeDtypeStruct((n, k_top), jnp.int32),
        ],
        scratch_shapes=[pltpu.VMEM((bn, n_e), jnp.float32)],
        compiler_params=pltpu.CompilerParams(
            dimension_semantics=("parallel", "arbitrary")),
    )(x, W1, b1r, W2, b2r)
    return (out, idx)
